# baseline (speedup 1.0000x reference)
import functools
import math

import jax
import jax.numpy as jnp
from jax import lax
from jax.experimental import pallas as pl
from jax.experimental.pallas import tpu as pltpu
from jax.experimental.pallas import tpu_sc as plsc

NC = 2
NS = 16
CHUNK = 128
CHUNK_E = 256
BN_EPS_ = 1e-5


def _pad_to(arr, n, fill):
    if arr.shape[0] == n:
        return arr
    return jnp.concatenate(
        [arr, jnp.full((n - arr.shape[0],) + arr.shape[1:], fill, arr.dtype)])


def _sc_gather(tab, idx, k_per_w):
    n_out = NC * NS * k_per_w * CHUNK
    d = tab.shape[1]
    idx3 = idx.reshape(NC, NS, k_per_w, CHUNK)
    mesh = plsc.VectorSubcoreMesh(core_axis_name="c", subcore_axis_name="s", num_cores=NC, num_subcores=NS)

    @functools.partial(
        pl.kernel,
        out_type=jax.ShapeDtypeStruct((n_out, d), jnp.float32),
        mesh=mesh,
        scratch_types=[
            pltpu.VMEM((CHUNK,), jnp.int32),
            pltpu.VMEM((CHUNK, d), jnp.float32),
            pltpu.SemaphoreType.DMA,
        ],
    )
    def gather_kernel(tab_hbm, idx_hbm, out_hbm, idx_v, rows_v, sem):
        c = lax.axis_index("c")
        s = lax.axis_index("s")
        base = (c * NS + s) * k_per_w * CHUNK

        def step(j, carry):
            pltpu.sync_copy(idx_hbm.at[c, s, j], idx_v)
            pltpu.async_copy(tab_hbm.at[idx_v], rows_v, sem).wait()
            pltpu.sync_copy(rows_v, out_hbm.at[pl.ds(base + j * CHUNK, CHUNK)])
            return carry

        lax.fori_loop(0, k_per_w, step, 0)

    return gather_kernel(tab, idx3)


def _sc_edge_aggregate(tab, src, dst, np_rows, k_per_w):
    d = tab.shape[1]
    rows_per_tile = np_rows // NS
    src3 = src.reshape(NC, NS, k_per_w, CHUNK_E)
    dst3 = dst.reshape(NC, NS, k_per_w, CHUNK_E)
    zeros = jnp.zeros((np_rows, d), jnp.float32)
    mesh = plsc.VectorSubcoreMesh(core_axis_name="c", subcore_axis_name="s", num_cores=NC, num_subcores=NS)

    @functools.partial(
        pl.kernel,
        out_type=jax.ShapeDtypeStruct((NC, np_rows, d), jnp.float32),
        mesh=mesh,
        scratch_types=[
            pltpu.VMEM_SHARED((np_rows, d), jnp.float32),
            pltpu.VMEM((CHUNK_E,), jnp.int32),
            pltpu.VMEM((CHUNK_E,), jnp.int32),
            pltpu.VMEM((CHUNK_E, d), jnp.float32),
            pltpu.SemaphoreType.DMA,
        ],
    )
    def agg_kernel(tab_hbm, src_hbm, dst_hbm, zero_hbm, parts_hbm,
                   acc, src_v, dst_v, rows_v, sem):
        c = lax.axis_index("c")
        s = lax.axis_index("s")
        r0 = s * rows_per_tile
        pltpu.sync_copy(zero_hbm.at[pl.ds(r0, rows_per_tile)],
                        acc.at[pl.ds(r0, rows_per_tile)])
        plsc.subcore_barrier()

        def step(j, carry):
            pltpu.sync_copy(src_hbm.at[c, s, j], src_v)
            pltpu.sync_copy(dst_hbm.at[c, s, j], dst_v)
            pltpu.async_copy(tab_hbm.at[src_v], rows_v, sem).wait()
            pltpu.sync_copy(rows_v, acc.at[dst_v], add=True)
            return carry

        lax.fori_loop(0, k_per_w, step, 0)
        plsc.subcore_barrier()
        pltpu.sync_copy(acc.at[pl.ds(r0, rows_per_tile)],
                        parts_hbm.at[c, pl.ds(r0, rows_per_tile)])

    return agg_kernel(tab, src3, dst3, zeros)


def _mlp_body(h_ref, a0_ref, a1_ref, wa_ref, ba_ref, wb_ref, bb_ref, out_ref):
    z = h_ref[...] + a0_ref[0] + a1_ref[0]
    t = jnp.dot(z, wa_ref[...], preferred_element_type=jnp.float32) + ba_ref[...]
    t = jnp.maximum(t, 0.0)
    u = jnp.dot(t, wb_ref[...], preferred_element_type=jnp.float32) + bb_ref[...]
    out_ref[...] = jnp.maximum(u, 0.0)


def _tc_mlp(h, parts, wa, ba, wb, bb, n, blk):
    d = h.shape[1]
    grid = n // blk
    return pl.pallas_call(
        _mlp_body,
        grid=(grid,),
        in_specs=[
            pl.BlockSpec((blk, d), lambda i: (i, 0)),
            pl.BlockSpec((1, blk, d), lambda i: (0, i, 0)),
            pl.BlockSpec((1, blk, d), lambda i: (1, i, 0)),
            pl.BlockSpec((d, d), lambda i: (0, 0)),
            pl.BlockSpec((1, d), lambda i: (0, 0)),
            pl.BlockSpec((d, d), lambda i: (0, 0)),
            pl.BlockSpec((1, d), lambda i: (0, 0)),
        ],
        out_specs=pl.BlockSpec((blk, d), lambda i: (i, 0)),
        out_shape=jax.ShapeDtypeStruct((n, d), jnp.float32),
    )(h, parts, parts, wa, ba, wb, bb)


def _mlp_pool_body(h_ref, a0_ref, a1_ref, wa_ref, ba_ref, wb_ref, bb_ref,
                   wl_ref, bl_ref, out_ref, acc_ref):
    i = pl.program_id(0)
    z = h_ref[...] + a0_ref[0] + a1_ref[0]
    t = jnp.dot(z, wa_ref[...], preferred_element_type=jnp.float32) + ba_ref[...]
    t = jnp.maximum(t, 0.0)
    u = jnp.dot(t, wb_ref[...], preferred_element_type=jnp.float32) + bb_ref[...]
    u = jnp.maximum(u, 0.0)
    bs = jnp.sum(u, axis=0, keepdims=True)

    @pl.when(i == 0)
    def _():
        acc_ref[...] = bs

    @pl.when(i > 0)
    def _():
        acc_ref[...] = acc_ref[...] + bs

    @pl.when(i == pl.num_programs(0) - 1)
    def _():
        out_ref[...] = (
            jnp.dot(acc_ref[...], wl_ref[...],
                    preferred_element_type=jnp.float32) + bl_ref[...])


def _tc_mlp_pool(h, parts, wa, ba, wb, bb, wl, bl, n, blk):
    d = h.shape[1]
    o = wl.shape[1]
    grid = n // blk
    return pl.pallas_call(
        _mlp_pool_body,
        grid=(grid,),
        in_specs=[
            pl.BlockSpec((blk, d), lambda i: (i, 0)),
            pl.BlockSpec((1, blk, d), lambda i: (0, i, 0)),
            pl.BlockSpec((1, blk, d), lambda i: (1, i, 0)),
            pl.BlockSpec((d, d), lambda i: (0, 0)),
            pl.BlockSpec((1, d), lambda i: (0, 0)),
            pl.BlockSpec((d, d), lambda i: (0, 0)),
            pl.BlockSpec((1, d), lambda i: (0, 0)),
            pl.BlockSpec((d, o), lambda i: (0, 0)),
            pl.BlockSpec((1, o), lambda i: (0, 0)),
        ],
        out_specs=pl.BlockSpec((1, o), lambda i: (0, 0)),
        out_shape=jax.ShapeDtypeStruct((1, o), jnp.float32),
        scratch_shapes=[pltpu.VMEM((1, d), jnp.float32)],
    )(h, parts, parts, wa, ba, wb, bb, wl, bl)


def kernel(x, edge_index, emb, W1a, b1a, g1a, be1a, W1b, b1b, g1b, be1b,
           W2a, b2a, g2a, be2a, W2b, b2b, g2b, be2b, Wl, bl):
    n, d = emb.shape
    e = edge_index.shape[1]
    scale = 1.0 / math.sqrt(1.0 + BN_EPS_)

    def fold(w, b, g, be):
        gs = g * scale
        return w * gs[None, :], (b * gs + be)[None, :]

    wa1, ba1 = fold(W1a, b1a, g1a, be1a)
    wb1, bb1 = fold(W1b, b1b, g1b, be1b)
    wa2, ba2 = fold(W2a, b2a, g2a, be2a)
    wb2, bb2 = fold(W2b, b2b, g2b, be2b)
    bl2 = bl[None, :]

    stride = NC * NS * CHUNK

    k_x = -(-n // stride)
    xi = _pad_to(x[:, 0].astype(jnp.int32), k_x * stride, 0)
    h0 = _sc_gather(emb, xi, k_x)

    np_rows = -(-n // (NS * 8)) * (NS * 8)
    stride_e = NC * NS * CHUNK_E
    k_e = -(-e // stride_e)
    src = _pad_to(edge_index[0].astype(jnp.int32), k_e * stride_e, 0)
    dst = _pad_to(edge_index[1].astype(jnp.int32), k_e * stride_e, n)

    parts1 = _sc_edge_aggregate(h0, src, dst, np_rows, k_e)
    blk = 1000
    h1 = _tc_mlp(h0, parts1, wa1, ba1, wb1, bb1, n, blk)
    parts2 = _sc_edge_aggregate(h1, src, dst, np_rows, k_e)
    return _tc_mlp_pool(h1, parts2, wa2, ba2, wb2, bb2, Wl, bl2, n, blk)


# speedup vs baseline: 1.5503x; 1.5503x over previous
import functools
import math

import jax
import jax.numpy as jnp
from jax import lax
from jax.experimental import pallas as pl
from jax.experimental.pallas import tpu as pltpu
from jax.experimental.pallas import tpu_sc as plsc

NC = 2
NS = 16
CHUNK = 128
BN_EPS_ = 1e-5


def _pad_to(arr, n, fill):
    if arr.shape[0] == n:
        return arr
    return jnp.concatenate(
        [arr, jnp.full((n - arr.shape[0],) + arr.shape[1:], fill, arr.dtype)])


def _sc_gather(tab, idx, k_per_w):
    n_out = NC * NS * k_per_w * CHUNK
    d = tab.shape[1]
    idx3 = idx.reshape(NC, NS, k_per_w, CHUNK)
    mesh = plsc.VectorSubcoreMesh(core_axis_name="c", subcore_axis_name="s", num_cores=NC, num_subcores=NS)

    @functools.partial(
        pl.kernel,
        out_type=jax.ShapeDtypeStruct((n_out, d), jnp.float32),
        mesh=mesh,
        scratch_types=[
            pltpu.VMEM((CHUNK,), jnp.int32),
            pltpu.VMEM((CHUNK, d), jnp.float32),
            pltpu.SemaphoreType.DMA,
        ],
    )
    def gather_kernel(tab_hbm, idx_hbm, out_hbm, idx_v, rows_v, sem):
        c = lax.axis_index("c")
        s = lax.axis_index("s")
        base = (c * NS + s) * k_per_w * CHUNK

        def step(j, carry):
            pltpu.sync_copy(idx_hbm.at[c, s, j], idx_v)
            pltpu.async_copy(tab_hbm.at[idx_v], rows_v, sem).wait()
            pltpu.sync_copy(rows_v, out_hbm.at[pl.ds(base + j * CHUNK, CHUNK)])
            return carry

        lax.fori_loop(0, k_per_w, step, 0)

    return gather_kernel(tab, idx3)


def _sc_edge_aggregate(tab, src, dst, np_rows, k_per_w):
    d = tab.shape[1]
    rows_per_tile = np_rows // NS
    src3 = src.reshape(NC, NS, k_per_w, 1, CHUNK)
    dst3 = dst.reshape(NC, NS, k_per_w, 1, CHUNK)
    zeros = jnp.zeros((np_rows, d), jnp.float32)
    mesh = plsc.VectorSubcoreMesh(core_axis_name="c", subcore_axis_name="s", num_cores=NC, num_subcores=NS)

    @functools.partial(
        pl.kernel,
        out_type=jax.ShapeDtypeStruct((NC, np_rows, d), jnp.float32),
        mesh=mesh,
        scratch_types=[
            pltpu.VMEM_SHARED((np_rows, d), jnp.float32),
            pltpu.VMEM((CHUNK, d), jnp.float32),
        ] + [pltpu.VMEM((CHUNK,), jnp.int32)] * (2 * k_per_w) + [
            pltpu.SemaphoreType.DMA,
        ],
    )
    def agg_kernel(tab_hbm, src_hbm, dst_hbm, zero_hbm, parts_hbm,
                   acc, rows_v, *rest):
        src_bufs = rest[:k_per_w]
        dst_bufs = rest[k_per_w:2 * k_per_w]
        sem = rest[2 * k_per_w]
        c = lax.axis_index("c")
        s = lax.axis_index("s")
        r0 = s * rows_per_tile

        for j in range(k_per_w):
            pltpu.async_copy(src_hbm.at[c, s, j, 0], src_bufs[j], sem)
            pltpu.async_copy(dst_hbm.at[c, s, j, 0], dst_bufs[j], sem)
        pltpu.sync_copy(zero_hbm.at[pl.ds(r0, rows_per_tile)],
                        acc.at[pl.ds(r0, rows_per_tile)])
        for j in range(k_per_w):
            pltpu.make_async_copy(src_hbm.at[c, s, j, 0], src_bufs[j], sem).wait()
            pltpu.make_async_copy(dst_hbm.at[c, s, j, 0], dst_bufs[j], sem).wait()
        plsc.subcore_barrier()

        for j in range(k_per_w):
            pltpu.async_copy(tab_hbm.at[src_bufs[j]], rows_v, sem).wait()
            pltpu.sync_copy(rows_v, acc.at[dst_bufs[j]], add=True)

        plsc.subcore_barrier()
        pltpu.sync_copy(acc.at[pl.ds(r0, rows_per_tile)],
                        parts_hbm.at[c, pl.ds(r0, rows_per_tile)])

    return agg_kernel(tab, src3, dst3, zeros)


def _mlp_body(h_ref, a0_ref, a1_ref, wa_ref, ba_ref, wb_ref, bb_ref, out_ref):
    z = h_ref[...] + a0_ref[0] + a1_ref[0]
    t = jnp.dot(z, wa_ref[...], preferred_element_type=jnp.float32) + ba_ref[...]
    t = jnp.maximum(t, 0.0)
    u = jnp.dot(t, wb_ref[...], preferred_element_type=jnp.float32) + bb_ref[...]
    out_ref[...] = jnp.maximum(u, 0.0)


def _tc_mlp(h, parts, wa, ba, wb, bb, n, blk):
    d = h.shape[1]
    grid = n // blk
    return pl.pallas_call(
        _mlp_body,
        grid=(grid,),
        in_specs=[
            pl.BlockSpec((blk, d), lambda i: (i, 0)),
            pl.BlockSpec((1, blk, d), lambda i: (0, i, 0)),
            pl.BlockSpec((1, blk, d), lambda i: (1, i, 0)),
            pl.BlockSpec((d, d), lambda i: (0, 0)),
            pl.BlockSpec((1, d), lambda i: (0, 0)),
            pl.BlockSpec((d, d), lambda i: (0, 0)),
            pl.BlockSpec((1, d), lambda i: (0, 0)),
        ],
        out_specs=pl.BlockSpec((blk, d), lambda i: (i, 0)),
        out_shape=jax.ShapeDtypeStruct((n, d), jnp.float32),
    )(h, parts, parts, wa, ba, wb, bb)


def _mlp_pool_body(h_ref, a0_ref, a1_ref, wa_ref, ba_ref, wb_ref, bb_ref,
                   wl_ref, bl_ref, out_ref, acc_ref):
    i = pl.program_id(0)
    z = h_ref[...] + a0_ref[0] + a1_ref[0]
    t = jnp.dot(z, wa_ref[...], preferred_element_type=jnp.float32) + ba_ref[...]
    t = jnp.maximum(t, 0.0)
    u = jnp.dot(t, wb_ref[...], preferred_element_type=jnp.float32) + bb_ref[...]
    u = jnp.maximum(u, 0.0)
    bs = jnp.sum(u, axis=0, keepdims=True)

    @pl.when(i == 0)
    def _():
        acc_ref[...] = bs

    @pl.when(i > 0)
    def _():
        acc_ref[...] = acc_ref[...] + bs

    @pl.when(i == pl.num_programs(0) - 1)
    def _():
        out_ref[...] = (
            jnp.dot(acc_ref[...], wl_ref[...],
                    preferred_element_type=jnp.float32) + bl_ref[...])


def _tc_mlp_pool(h, parts, wa, ba, wb, bb, wl, bl, n, blk):
    d = h.shape[1]
    o = wl.shape[1]
    grid = n // blk
    return pl.pallas_call(
        _mlp_pool_body,
        grid=(grid,),
        in_specs=[
            pl.BlockSpec((blk, d), lambda i: (i, 0)),
            pl.BlockSpec((1, blk, d), lambda i: (0, i, 0)),
            pl.BlockSpec((1, blk, d), lambda i: (1, i, 0)),
            pl.BlockSpec((d, d), lambda i: (0, 0)),
            pl.BlockSpec((1, d), lambda i: (0, 0)),
            pl.BlockSpec((d, d), lambda i: (0, 0)),
            pl.BlockSpec((1, d), lambda i: (0, 0)),
            pl.BlockSpec((d, o), lambda i: (0, 0)),
            pl.BlockSpec((1, o), lambda i: (0, 0)),
        ],
        out_specs=pl.BlockSpec((1, o), lambda i: (0, 0)),
        out_shape=jax.ShapeDtypeStruct((1, o), jnp.float32),
        scratch_shapes=[pltpu.VMEM((1, d), jnp.float32)],
    )(h, parts, parts, wa, ba, wb, bb, wl, bl)


def kernel(x, edge_index, emb, W1a, b1a, g1a, be1a, W1b, b1b, g1b, be1b,
           W2a, b2a, g2a, be2a, W2b, b2b, g2b, be2b, Wl, bl):
    n, d = emb.shape
    e = edge_index.shape[1]
    scale = 1.0 / math.sqrt(1.0 + BN_EPS_)

    def fold(w, b, g, be):
        gs = g * scale
        return w * gs[None, :], (b * gs + be)[None, :]

    wa1, ba1 = fold(W1a, b1a, g1a, be1a)
    wb1, bb1 = fold(W1b, b1b, g1b, be1b)
    wa2, ba2 = fold(W2a, b2a, g2a, be2a)
    wb2, bb2 = fold(W2b, b2b, g2b, be2b)
    bl2 = bl[None, :]

    stride = NC * NS * CHUNK

    k_x = -(-n // stride)
    xi = _pad_to(x[:, 0].astype(jnp.int32), k_x * stride, 0)
    h0 = _sc_gather(emb, xi, k_x)

    np_rows = -(-n // (NS * 8)) * (NS * 8)
    k_e = -(-e // stride)
    src = _pad_to(edge_index[0].astype(jnp.int32), k_e * stride, 0)
    dst = _pad_to(edge_index[1].astype(jnp.int32), k_e * stride, n)

    parts1 = _sc_edge_aggregate(h0, src, dst, np_rows, k_e)
    blk = 1000
    h1 = _tc_mlp(h0, parts1, wa1, ba1, wb1, bb1, n, blk)
    parts2 = _sc_edge_aggregate(h1, src, dst, np_rows, k_e)
    return _tc_mlp_pool(h1, parts2, wa2, ba2, wb2, bb2, Wl, bl2, n, blk)
